# baseline (speedup 1.0000x reference)
import functools

import jax
import jax.numpy as jnp
from jax import lax
from jax.experimental import pallas as pl
from jax.experimental.pallas import tpu as pltpu
from jax.experimental.pallas import tpu_sc as plsc

B = 8
C = 3
H = 224
P = 4
HIDDEN = 256
D = 256
K = 1024
PD = C * P * P
HP = H // P
WP = 224 // P
N = B * HP * WP
NH = N // 2
COMMITMENT = 0.25
DATA_VAR = 1.0

TB_A = 896
NB_A = NH // TB_A
RB_C = 28
TB_C = RB_C * WP
NR_C = HP // RB_C
NB_C = NH // TB_C

SC_NW = 32
SC_ROWS = NH // SC_NW


def _patchify(x):
    eye = jnp.eye(PD, dtype=x.dtype).reshape(PD, C, P, P)
    dn = lax.conv_dimension_numbers(x.shape, eye.shape, ("NCHW", "OIHW", "NHWC"))
    return lax.conv_general_dilated(x, eye, (P, P), "VALID", dimension_numbers=dn)


def _unpatchify(d):
    eye = jnp.eye(PD, dtype=d.dtype).reshape(C, P, P, PD).transpose(1, 2, 3, 0)
    eye = eye[::-1, ::-1]
    return lax.conv_transpose(d, eye, (P, P), "VALID",
                              dimension_numbers=("NHWC", "HWIO", "NCHW"))


def _enc_vq_body(p_ref, we_ref, be_ref, wp_ref, bp_ref, emb_ref,
                 z_ref, idx_ref, counts_ref):
    i = pl.program_id(0)
    h = jnp.maximum(
        lax.dot_general(p_ref[...], we_ref[...], (((0,), (0,)), ((), ())))
        + be_ref[...], 0.0)
    z = jnp.dot(h, wp_ref[...]) + bp_ref[...]
    z_ref[...] = z
    emb = emb_ref[...]
    zsq = jnp.sum(z * z, axis=1, keepdims=True)
    esq = jnp.sum(emb * emb, axis=1)
    mm2 = lax.dot_general(2.0 * z, emb, (((1,), (1,)), ((), ())))
    dist = (zsq + esq) - mm2
    m = jnp.min(dist, axis=1, keepdims=True)
    ks = lax.broadcasted_iota(jnp.int32, dist.shape, 1)
    idx = jnp.min(jnp.where(dist == m, ks, K), axis=1).astype(jnp.int32)
    idx_ref[0, 0, :] = idx
    onehot = (idx[:, None] == lax.broadcasted_iota(jnp.int32, (TB_A, K), 1))
    part = jnp.sum(onehot.astype(jnp.float32), axis=0, keepdims=True)

    @pl.when(i == 0)
    def _():
        counts_ref[...] = jnp.zeros_like(counts_ref)

    counts_ref[...] += part


def _enc_vq(patches, W_enc, b_enc, W_pre, b_pre, emb, half):
    off = half * NB_A
    return pl.pallas_call(
        _enc_vq_body,
        grid=(NB_A,),
        in_specs=[
            pl.BlockSpec((PD, TB_A), lambda i: (0, i + off)),
            pl.BlockSpec((PD, HIDDEN), lambda i: (0, 0)),
            pl.BlockSpec((1, HIDDEN), lambda i: (0, 0)),
            pl.BlockSpec((HIDDEN, D), lambda i: (0, 0)),
            pl.BlockSpec((1, D), lambda i: (0, 0)),
            pl.BlockSpec((K, D), lambda i: (0, 0)),
        ],
        out_specs=[
            pl.BlockSpec((TB_A, D), lambda i: (i, 0)),
            pl.BlockSpec((1, 1, TB_A), lambda i: (i, 0, 0)),
            pl.BlockSpec((1, K), lambda i: (0, 0)),
        ],
        out_shape=[
            jax.ShapeDtypeStruct((NH, D), jnp.float32),
            jax.ShapeDtypeStruct((NB_A, 1, TB_A), jnp.int32),
            jax.ShapeDtypeStruct((1, K), jnp.float32),
        ],
    )(patches, W_enc, b_enc, W_pre, b_pre, emb)


def _sc_gather_body(emb_hbm, idx_hbm, out_hbm, idx_v, rows_v, sem):
    c = lax.axis_index("c")
    s = lax.axis_index("s")
    base = (s * 2 + c) * SC_ROWS
    pltpu.sync_copy(idx_hbm.at[pl.ds(base, SC_ROWS)], idx_v)
    pltpu.async_copy(emb_hbm.at[idx_v], rows_v, sem).wait()
    pltpu.sync_copy(rows_v, out_hbm.at[pl.ds(base, SC_ROWS)])


@functools.cache
def _sc_gather_kernel():
    return pl.kernel(
        _sc_gather_body,
        out_type=jax.ShapeDtypeStruct((NH, D), jnp.float32),
        mesh=plsc.VectorSubcoreMesh(core_axis_name="c", subcore_axis_name="s"),
        scratch_types=[
            pltpu.VMEM((SC_ROWS,), jnp.int32),
            pltpu.VMEM((SC_ROWS, D), jnp.float32),
            pltpu.SemaphoreType.DMA,
        ],
    )


def _sc_gather(emb, idx):
    return _sc_gather_kernel()(emb, idx)


def _dec_loss_body(x_ref, z_ref, q_ref, wd_ref, bd_ref, c1_ref, c2_ref,
                   a1_ref, a2_ref,
                   qst_ref, xrec_ref, loss_ref, perp_ref, rec_ref, vq_ref,
                   acc_ref, *, final):
    i = pl.program_id(0) * NR_C + pl.program_id(1)
    z = z_ref[...]
    q = q_ref[...]
    qst = z + (q - z)
    qst_ref[...] = qst
    dec = jnp.dot(qst, wd_ref[...]) + bd_ref[...]
    xr = dec.reshape(RB_C, WP, C, P, P).transpose(2, 0, 3, 1, 4).reshape(C, RB_C * P, H)
    xrec_ref[...] = xr[None]

    @pl.when(i == 0)
    def _():
        acc_ref[0] = 0.0
        acc_ref[1] = 0.0

    acc_ref[0] += jnp.sum((q - z) ** 2)
    acc_ref[1] += jnp.sum((xr - x_ref[0]) ** 2)

    @pl.when(i == NB_C - 1)
    def _():
        vq_sum = acc_ref[0] + a1_ref[0, 0]
        rec_sum = acc_ref[1] + a2_ref[0, 0]
        if final:
            latent = vq_sum / (N * D)
            vq = latent + COMMITMENT * latent
            rec = rec_sum / (N * PD) / DATA_VAR
            p = (c1_ref[...] + c2_ref[...]) / N
            ent = jnp.sum(p * jnp.log(p + 1e-10))
            perp_ref[...] = jnp.exp(-ent).reshape(1, 1)
            vq_ref[...] = vq.reshape(1, 1)
            rec_ref[...] = rec.reshape(1, 1)
            loss_ref[...] = (rec + vq).reshape(1, 1)
        else:
            perp_ref[...] = jnp.zeros((1, 1), jnp.float32)
            vq_ref[...] = vq_sum.reshape(1, 1)
            rec_ref[...] = rec_sum.reshape(1, 1)
            loss_ref[...] = jnp.zeros((1, 1), jnp.float32)


def _dec_loss(x, z, q, W_dec, b_dec, c1, c2, a1, a2, half, final):
    boff = half * (B // 2)
    return pl.pallas_call(
        functools.partial(_dec_loss_body, final=final),
        grid=(B // 2, NR_C),
        in_specs=[
            pl.BlockSpec((1, C, RB_C * P, H), lambda b, r: (b + boff, 0, r, 0)),
            pl.BlockSpec((TB_C, D), lambda b, r: (b * NR_C + r, 0)),
            pl.BlockSpec((TB_C, D), lambda b, r: (b * NR_C + r, 0)),
            pl.BlockSpec((D, PD), lambda b, r: (0, 0)),
            pl.BlockSpec((1, PD), lambda b, r: (0, 0)),
            pl.BlockSpec((1, K), lambda b, r: (0, 0)),
            pl.BlockSpec((1, K), lambda b, r: (0, 0)),
            pl.BlockSpec((1, 1), lambda b, r: (0, 0)),
            pl.BlockSpec((1, 1), lambda b, r: (0, 0)),
        ],
        out_specs=[
            pl.BlockSpec((TB_C, D), lambda b, r: (b * NR_C + r, 0)),
            pl.BlockSpec((1, C, RB_C * P, H), lambda b, r: (b, 0, r, 0)),
            pl.BlockSpec((1, 1), lambda b, r: (0, 0)),
            pl.BlockSpec((1, 1), lambda b, r: (0, 0)),
            pl.BlockSpec((1, 1), lambda b, r: (0, 0)),
            pl.BlockSpec((1, 1), lambda b, r: (0, 0)),
        ],
        out_shape=[
            jax.ShapeDtypeStruct((NH, D), jnp.float32),
            jax.ShapeDtypeStruct((B // 2, C, H, H), jnp.float32),
            jax.ShapeDtypeStruct((1, 1), jnp.float32),
            jax.ShapeDtypeStruct((1, 1), jnp.float32),
            jax.ShapeDtypeStruct((1, 1), jnp.float32),
            jax.ShapeDtypeStruct((1, 1), jnp.float32),
        ],
        scratch_shapes=[pltpu.SMEM((2,), jnp.float32)],
    )(x, z, q, W_dec, b_dec, c1, c2, a1, a2)


def kernel(inputs, W_enc, b_enc, W_pre, b_pre, W_dec, b_dec, emb):
    patches = _patchify(inputs).reshape(N, PD).T
    be = b_enc.reshape(1, HIDDEN)
    bp = b_pre.reshape(1, D)
    bd = b_dec.reshape(1, PD)
    zero = jnp.zeros((1, 1), jnp.float32)

    z1, idx31, counts1 = _enc_vq(patches, W_enc, be, W_pre, bp, emb, 0)
    q1 = _sc_gather(emb, idx31.reshape(NH))
    z2, idx32, counts2 = _enc_vq(patches, W_enc, be, W_pre, bp, emb, 1)
    q2 = _sc_gather(emb, idx32.reshape(NH))
    qst1, xr1, _, _, rsum1, vsum1 = _dec_loss(
        inputs, z1, q1, W_dec, bd, counts1, counts2, zero, zero, 0, False)
    qst2, xr2, loss, perp, rec, vq = _dec_loss(
        inputs, z2, q2, W_dec, bd, counts1, counts2, vsum1, rsum1, 1, True)

    x_rec = jnp.concatenate([xr1, xr2], axis=0)
    qst = jnp.concatenate([qst1, qst2], axis=0)
    return (loss.reshape(()), x_rec, qst.reshape(B, HP, WP, D),
            perp.reshape(()), rec.reshape(()), vq.reshape(()))


# speedup vs baseline: 1.0418x; 1.0418x over previous
import functools

import jax
import jax.numpy as jnp
from jax import lax
from jax.experimental import pallas as pl
from jax.experimental.pallas import tpu as pltpu
from jax.experimental.pallas import tpu_sc as plsc

B = 8
C = 3
H = 224
P = 4
HIDDEN = 256
D = 256
K = 1024
PD = C * P * P
HP = H // P
WP = 224 // P
N = B * HP * WP
NH = N // 2
COMMITMENT = 0.25
DATA_VAR = 1.0

TB_A = 1568
NB_A = NH // TB_A
RB_C = 28
TB_C = RB_C * WP
NR_C = HP // RB_C
NB_C = NH // TB_C

SC_NW = 32
SC_ROWS = NH // SC_NW


def _patchify(x):
    eye = jnp.eye(PD, dtype=x.dtype).reshape(PD, C, P, P)
    dn = lax.conv_dimension_numbers(x.shape, eye.shape, ("NCHW", "OIHW", "NHWC"))
    return lax.conv_general_dilated(x, eye, (P, P), "VALID", dimension_numbers=dn)


def _unpatchify(d):
    eye = jnp.eye(PD, dtype=d.dtype).reshape(C, P, P, PD).transpose(1, 2, 3, 0)
    eye = eye[::-1, ::-1]
    return lax.conv_transpose(d, eye, (P, P), "VALID",
                              dimension_numbers=("NHWC", "HWIO", "NCHW"))


def _enc_vq_body(p_ref, we_ref, be_ref, wp_ref, bp_ref, emb_ref,
                 z_ref, idx_ref, counts_ref):
    i = pl.program_id(0)
    h = jnp.maximum(jnp.dot(p_ref[...], we_ref[...]) + be_ref[...], 0.0)
    z = jnp.dot(h, wp_ref[...]) + bp_ref[...]
    z_ref[...] = z
    emb = emb_ref[...]
    zsq = jnp.sum(z * z, axis=1, keepdims=True)
    esq = jnp.sum(emb * emb, axis=1)
    mm2 = lax.dot_general(2.0 * z, emb, (((1,), (1,)), ((), ())))
    dist = (zsq + esq) - mm2
    m = jnp.min(dist, axis=1, keepdims=True)
    ks = lax.broadcasted_iota(jnp.int32, dist.shape, 1)
    idx = jnp.min(jnp.where(dist == m, ks, K), axis=1).astype(jnp.int32)
    idx_ref[0, 0, :] = idx
    onehot = (idx[:, None] == lax.broadcasted_iota(jnp.int32, (TB_A, K), 1))
    part = jnp.sum(onehot.astype(jnp.float32), axis=0, keepdims=True)

    @pl.when(i == 0)
    def _():
        counts_ref[...] = jnp.zeros_like(counts_ref)

    counts_ref[...] += part


def _enc_vq(patches, W_enc, b_enc, W_pre, b_pre, emb, half):
    off = half * NB_A
    return pl.pallas_call(
        _enc_vq_body,
        grid=(NB_A,),
        in_specs=[
            pl.BlockSpec((TB_A, PD), lambda i: (i + off, 0)),
            pl.BlockSpec((PD, HIDDEN), lambda i: (0, 0)),
            pl.BlockSpec((1, HIDDEN), lambda i: (0, 0)),
            pl.BlockSpec((HIDDEN, D), lambda i: (0, 0)),
            pl.BlockSpec((1, D), lambda i: (0, 0)),
            pl.BlockSpec((K, D), lambda i: (0, 0)),
        ],
        out_specs=[
            pl.BlockSpec((TB_A, D), lambda i: (i, 0)),
            pl.BlockSpec((1, 1, TB_A), lambda i: (i, 0, 0)),
            pl.BlockSpec((1, K), lambda i: (0, 0)),
        ],
        out_shape=[
            jax.ShapeDtypeStruct((NH, D), jnp.float32),
            jax.ShapeDtypeStruct((NB_A, 1, TB_A), jnp.int32),
            jax.ShapeDtypeStruct((1, K), jnp.float32),
        ],
    )(patches, W_enc, b_enc, W_pre, b_pre, emb)


def _sc_gather_body(emb_hbm, idx_hbm, out_hbm, idx_v, rows_v, sem):
    c = lax.axis_index("c")
    s = lax.axis_index("s")
    base = (s * 2 + c) * SC_ROWS
    pltpu.sync_copy(idx_hbm.at[pl.ds(base, SC_ROWS)], idx_v)
    pltpu.async_copy(emb_hbm.at[idx_v], rows_v, sem).wait()
    pltpu.sync_copy(rows_v, out_hbm.at[pl.ds(base, SC_ROWS)])


@functools.cache
def _sc_gather_kernel():
    return pl.kernel(
        _sc_gather_body,
        out_type=jax.ShapeDtypeStruct((NH, D), jnp.float32),
        mesh=plsc.VectorSubcoreMesh(core_axis_name="c", subcore_axis_name="s"),
        scratch_types=[
            pltpu.VMEM((SC_ROWS,), jnp.int32),
            pltpu.VMEM((SC_ROWS, D), jnp.float32),
            pltpu.SemaphoreType.DMA,
        ],
    )


def _sc_gather(emb, idx):
    return _sc_gather_kernel()(emb, idx)


def _dec_loss_body(x_ref, z_ref, q_ref, wd_ref, bd_ref, c1_ref, c2_ref,
                   a1_ref, a2_ref,
                   qst_ref, xrec_ref, loss_ref, perp_ref, rec_ref, vq_ref,
                   acc_ref, *, final):
    i = pl.program_id(0) * NR_C + pl.program_id(1)
    z = z_ref[...]
    q = q_ref[...]
    qst = z + (q - z)
    qst_ref[...] = qst
    dec = jnp.dot(qst, wd_ref[...]) + bd_ref[...]
    xr = dec.reshape(RB_C, WP, C, P, P).transpose(2, 0, 3, 1, 4).reshape(C, RB_C * P, H)
    xrec_ref[...] = xr[None]

    @pl.when(i == 0)
    def _():
        acc_ref[0] = 0.0
        acc_ref[1] = 0.0

    acc_ref[0] += jnp.sum((q - z) ** 2)
    acc_ref[1] += jnp.sum((xr - x_ref[0]) ** 2)

    @pl.when(i == NB_C - 1)
    def _():
        vq_sum = acc_ref[0] + a1_ref[0, 0]
        rec_sum = acc_ref[1] + a2_ref[0, 0]
        if final:
            latent = vq_sum / (N * D)
            vq = latent + COMMITMENT * latent
            rec = rec_sum / (N * PD) / DATA_VAR
            p = (c1_ref[...] + c2_ref[...]) / N
            ent = jnp.sum(p * jnp.log(p + 1e-10))
            perp_ref[...] = jnp.exp(-ent).reshape(1, 1)
            vq_ref[...] = vq.reshape(1, 1)
            rec_ref[...] = rec.reshape(1, 1)
            loss_ref[...] = (rec + vq).reshape(1, 1)
        else:
            perp_ref[...] = jnp.zeros((1, 1), jnp.float32)
            vq_ref[...] = vq_sum.reshape(1, 1)
            rec_ref[...] = rec_sum.reshape(1, 1)
            loss_ref[...] = jnp.zeros((1, 1), jnp.float32)


def _dec_loss(x, z, q, W_dec, b_dec, c1, c2, a1, a2, half, final):
    boff = half * (B // 2)
    return pl.pallas_call(
        functools.partial(_dec_loss_body, final=final),
        grid=(B // 2, NR_C),
        in_specs=[
            pl.BlockSpec((1, C, RB_C * P, H), lambda b, r: (b + boff, 0, r, 0)),
            pl.BlockSpec((TB_C, D), lambda b, r: (b * NR_C + r, 0)),
            pl.BlockSpec((TB_C, D), lambda b, r: (b * NR_C + r, 0)),
            pl.BlockSpec((D, PD), lambda b, r: (0, 0)),
            pl.BlockSpec((1, PD), lambda b, r: (0, 0)),
            pl.BlockSpec((1, K), lambda b, r: (0, 0)),
            pl.BlockSpec((1, K), lambda b, r: (0, 0)),
            pl.BlockSpec((1, 1), lambda b, r: (0, 0)),
            pl.BlockSpec((1, 1), lambda b, r: (0, 0)),
        ],
        out_specs=[
            pl.BlockSpec((TB_C, D), lambda b, r: (b * NR_C + r, 0)),
            pl.BlockSpec((1, C, RB_C * P, H), lambda b, r: (b, 0, r, 0)),
            pl.BlockSpec((1, 1), lambda b, r: (0, 0)),
            pl.BlockSpec((1, 1), lambda b, r: (0, 0)),
            pl.BlockSpec((1, 1), lambda b, r: (0, 0)),
            pl.BlockSpec((1, 1), lambda b, r: (0, 0)),
        ],
        out_shape=[
            jax.ShapeDtypeStruct((NH, D), jnp.float32),
            jax.ShapeDtypeStruct((B // 2, C, H, H), jnp.float32),
            jax.ShapeDtypeStruct((1, 1), jnp.float32),
            jax.ShapeDtypeStruct((1, 1), jnp.float32),
            jax.ShapeDtypeStruct((1, 1), jnp.float32),
            jax.ShapeDtypeStruct((1, 1), jnp.float32),
        ],
        scratch_shapes=[pltpu.SMEM((2,), jnp.float32)],
    )(x, z, q, W_dec, b_dec, c1, c2, a1, a2)


def kernel(inputs, W_enc, b_enc, W_pre, b_pre, W_dec, b_dec, emb):
    patches = _patchify(inputs).reshape(N, PD)
    be = b_enc.reshape(1, HIDDEN)
    bp = b_pre.reshape(1, D)
    bd = b_dec.reshape(1, PD)
    zero = jnp.zeros((1, 1), jnp.float32)

    z1, idx31, counts1 = _enc_vq(patches, W_enc, be, W_pre, bp, emb, 0)
    q1 = _sc_gather(emb, idx31.reshape(NH))
    z2, idx32, counts2 = _enc_vq(patches, W_enc, be, W_pre, bp, emb, 1)
    q2 = _sc_gather(emb, idx32.reshape(NH))
    qst1, xr1, _, _, rsum1, vsum1 = _dec_loss(
        inputs, z1, q1, W_dec, bd, counts1, counts2, zero, zero, 0, False)
    qst2, xr2, loss, perp, rec, vq = _dec_loss(
        inputs, z2, q2, W_dec, bd, counts1, counts2, vsum1, rsum1, 1, True)

    x_rec = jnp.concatenate([xr1, xr2], axis=0)
    qst = jnp.concatenate([qst1, qst2], axis=0)
    return (loss.reshape(()), x_rec, qst.reshape(B, HP, WP, D),
            perp.reshape(()), rec.reshape(()), vq.reshape(()))


# speedup vs baseline: 1.0451x; 1.0032x over previous
import functools

import jax
import jax.numpy as jnp
from jax import lax
from jax.experimental import pallas as pl
from jax.experimental.pallas import tpu as pltpu
from jax.experimental.pallas import tpu_sc as plsc

B = 8
C = 3
H = 224
P = 4
HIDDEN = 256
D = 256
K = 1024
PD = C * P * P
HP = H // P
WP = 224 // P
N = B * HP * WP
NH = N // 2
COMMITMENT = 0.25
DATA_VAR = 1.0

TB_A = 1568
NB_A = NH // TB_A
RB_C = 28
TB_C = RB_C * WP
NR_C = HP // RB_C
NB_C = NH // TB_C

SC_NW = 32
SC_ROWS = NH // SC_NW


def _patchify(x):
    eye = jnp.eye(PD, dtype=x.dtype).reshape(PD, C, P, P)
    dn = lax.conv_dimension_numbers(x.shape, eye.shape, ("NCHW", "OIHW", "NHWC"))
    return lax.conv_general_dilated(x, eye, (P, P), "VALID", dimension_numbers=dn)


def _unpatchify(d):
    eye = jnp.eye(PD, dtype=d.dtype).reshape(C, P, P, PD).transpose(1, 2, 3, 0)
    eye = eye[::-1, ::-1]
    return lax.conv_transpose(d, eye, (P, P), "VALID",
                              dimension_numbers=("NHWC", "HWIO", "NCHW"))


def _enc_vq_body(p_ref, we_ref, be_ref, wp_ref, bp_ref, emb_ref,
                 z_ref, idx_ref, counts_ref):
    i = pl.program_id(0)
    h = jnp.maximum(jnp.dot(p_ref[...], we_ref[...]) + be_ref[...], 0.0)
    z = jnp.dot(h, wp_ref[...]) + bp_ref[...]
    z_ref[...] = z
    emb = emb_ref[...]
    z2 = z * z
    zsq = (jnp.sum(z2[:, :128], axis=1, keepdims=True)
           + jnp.sum(z2[:, 128:], axis=1, keepdims=True))
    esq = jnp.sum(emb * emb, axis=1)
    mm2 = lax.dot_general(2.0 * z, emb, (((1,), (1,)), ((), ())))
    dist = (zsq + esq) - mm2
    m = jnp.min(dist, axis=1, keepdims=True)
    ks = lax.broadcasted_iota(jnp.int32, dist.shape, 1)
    idx = jnp.min(jnp.where(dist == m, ks, K), axis=1).astype(jnp.int32)
    idx_ref[0, 0, :] = idx
    onehot = (idx[:, None] == lax.broadcasted_iota(jnp.int32, (TB_A, K), 1))
    part = jnp.sum(onehot.astype(jnp.float32), axis=0, keepdims=True)

    @pl.when(i == 0)
    def _():
        counts_ref[...] = jnp.zeros_like(counts_ref)

    counts_ref[...] += part


def _enc_vq(patches, W_enc, b_enc, W_pre, b_pre, emb, half):
    off = half * NB_A
    return pl.pallas_call(
        _enc_vq_body,
        grid=(NB_A,),
        in_specs=[
            pl.BlockSpec((TB_A, PD), lambda i: (i + off, 0)),
            pl.BlockSpec((PD, HIDDEN), lambda i: (0, 0)),
            pl.BlockSpec((1, HIDDEN), lambda i: (0, 0)),
            pl.BlockSpec((HIDDEN, D), lambda i: (0, 0)),
            pl.BlockSpec((1, D), lambda i: (0, 0)),
            pl.BlockSpec((K, D), lambda i: (0, 0)),
        ],
        out_specs=[
            pl.BlockSpec((TB_A, D), lambda i: (i, 0)),
            pl.BlockSpec((1, 1, TB_A), lambda i: (i, 0, 0)),
            pl.BlockSpec((1, K), lambda i: (0, 0)),
        ],
        out_shape=[
            jax.ShapeDtypeStruct((NH, D), jnp.float32),
            jax.ShapeDtypeStruct((NB_A, 1, TB_A), jnp.int32),
            jax.ShapeDtypeStruct((1, K), jnp.float32),
        ],
    )(patches, W_enc, b_enc, W_pre, b_pre, emb)


def _sc_gather_body(emb_hbm, idx_hbm, out_hbm, idx_v, rows_v, sem):
    c = lax.axis_index("c")
    s = lax.axis_index("s")
    base = (s * 2 + c) * SC_ROWS
    pltpu.sync_copy(idx_hbm.at[pl.ds(base, SC_ROWS)], idx_v)
    pltpu.async_copy(emb_hbm.at[idx_v], rows_v, sem).wait()
    pltpu.sync_copy(rows_v, out_hbm.at[pl.ds(base, SC_ROWS)])


@functools.cache
def _sc_gather_kernel():
    return pl.kernel(
        _sc_gather_body,
        out_type=jax.ShapeDtypeStruct((NH, D), jnp.float32),
        mesh=plsc.VectorSubcoreMesh(core_axis_name="c", subcore_axis_name="s"),
        scratch_types=[
            pltpu.VMEM((SC_ROWS,), jnp.int32),
            pltpu.VMEM((SC_ROWS, D), jnp.float32),
            pltpu.SemaphoreType.DMA,
        ],
    )


def _sc_gather(emb, idx):
    return _sc_gather_kernel()(emb, idx)


def _dec_loss_body(x_ref, z_ref, q_ref, wd_ref, bd_ref, c1_ref, c2_ref,
                   a1_ref, a2_ref,
                   qst_ref, xrec_ref, loss_ref, perp_ref, rec_ref, vq_ref,
                   acc_ref, *, final):
    i = pl.program_id(0) * NR_C + pl.program_id(1)
    z = z_ref[...]
    q = q_ref[...]
    qst = z + (q - z)
    qst_ref[...] = qst
    dec = jnp.dot(qst, wd_ref[...]) + bd_ref[...]
    xr = dec.reshape(RB_C, WP, C, P, P).transpose(2, 0, 3, 1, 4).reshape(C, RB_C * P, H)
    xrec_ref[...] = xr[None]

    @pl.when(i == 0)
    def _():
        acc_ref[0] = 0.0
        acc_ref[1] = 0.0

    acc_ref[0] += jnp.sum((q - z) ** 2)
    acc_ref[1] += jnp.sum((xr - x_ref[0]) ** 2)

    @pl.when(i == NB_C - 1)
    def _():
        vq_sum = acc_ref[0] + a1_ref[0, 0]
        rec_sum = acc_ref[1] + a2_ref[0, 0]
        if final:
            latent = vq_sum / (N * D)
            vq = latent + COMMITMENT * latent
            rec = rec_sum / (N * PD) / DATA_VAR
            p = (c1_ref[...] + c2_ref[...]) / N
            ent = jnp.sum(p * jnp.log(p + 1e-10))
            perp_ref[...] = jnp.exp(-ent).reshape(1, 1)
            vq_ref[...] = vq.reshape(1, 1)
            rec_ref[...] = rec.reshape(1, 1)
            loss_ref[...] = (rec + vq).reshape(1, 1)
        else:
            perp_ref[...] = jnp.zeros((1, 1), jnp.float32)
            vq_ref[...] = vq_sum.reshape(1, 1)
            rec_ref[...] = rec_sum.reshape(1, 1)
            loss_ref[...] = jnp.zeros((1, 1), jnp.float32)


def _dec_loss(x, z, q, W_dec, b_dec, c1, c2, a1, a2, half, final):
    boff = half * (B // 2)
    return pl.pallas_call(
        functools.partial(_dec_loss_body, final=final),
        grid=(B // 2, NR_C),
        in_specs=[
            pl.BlockSpec((1, C, RB_C * P, H), lambda b, r: (b + boff, 0, r, 0)),
            pl.BlockSpec((TB_C, D), lambda b, r: (b * NR_C + r, 0)),
            pl.BlockSpec((TB_C, D), lambda b, r: (b * NR_C + r, 0)),
            pl.BlockSpec((D, PD), lambda b, r: (0, 0)),
            pl.BlockSpec((1, PD), lambda b, r: (0, 0)),
            pl.BlockSpec((1, K), lambda b, r: (0, 0)),
            pl.BlockSpec((1, K), lambda b, r: (0, 0)),
            pl.BlockSpec((1, 1), lambda b, r: (0, 0)),
            pl.BlockSpec((1, 1), lambda b, r: (0, 0)),
        ],
        out_specs=[
            pl.BlockSpec((TB_C, D), lambda b, r: (b * NR_C + r, 0)),
            pl.BlockSpec((1, C, RB_C * P, H), lambda b, r: (b, 0, r, 0)),
            pl.BlockSpec((1, 1), lambda b, r: (0, 0)),
            pl.BlockSpec((1, 1), lambda b, r: (0, 0)),
            pl.BlockSpec((1, 1), lambda b, r: (0, 0)),
            pl.BlockSpec((1, 1), lambda b, r: (0, 0)),
        ],
        out_shape=[
            jax.ShapeDtypeStruct((NH, D), jnp.float32),
            jax.ShapeDtypeStruct((B // 2, C, H, H), jnp.float32),
            jax.ShapeDtypeStruct((1, 1), jnp.float32),
            jax.ShapeDtypeStruct((1, 1), jnp.float32),
            jax.ShapeDtypeStruct((1, 1), jnp.float32),
            jax.ShapeDtypeStruct((1, 1), jnp.float32),
        ],
        scratch_shapes=[pltpu.SMEM((2,), jnp.float32)],
    )(x, z, q, W_dec, b_dec, c1, c2, a1, a2)


def kernel(inputs, W_enc, b_enc, W_pre, b_pre, W_dec, b_dec, emb):
    patches = _patchify(inputs).reshape(N, PD)
    be = b_enc.reshape(1, HIDDEN)
    bp = b_pre.reshape(1, D)
    bd = b_dec.reshape(1, PD)
    zero = jnp.zeros((1, 1), jnp.float32)

    z1, idx31, counts1 = _enc_vq(patches, W_enc, be, W_pre, bp, emb, 0)
    q1 = _sc_gather(emb, idx31.reshape(NH))
    z2, idx32, counts2 = _enc_vq(patches, W_enc, be, W_pre, bp, emb, 1)
    q2 = _sc_gather(emb, idx32.reshape(NH))
    qst1, xr1, _, _, rsum1, vsum1 = _dec_loss(
        inputs, z1, q1, W_dec, bd, counts1, counts2, zero, zero, 0, False)
    qst2, xr2, loss, perp, rec, vq = _dec_loss(
        inputs, z2, q2, W_dec, bd, counts1, counts2, vsum1, rsum1, 1, True)

    x_rec = jnp.concatenate([xr1, xr2], axis=0)
    qst = jnp.concatenate([qst1, qst2], axis=0)
    return (loss.reshape(()), x_rec, qst.reshape(B, HP, WP, D),
            perp.reshape(()), rec.reshape(()), vq.reshape(()))
